# baseline (speedup 1.0000x reference)
import functools

import jax
import jax.numpy as jnp
from jax import lax
from jax.experimental import pallas as pl
from jax.experimental.pallas import tpu as pltpu
from jax.experimental.pallas import tpu_sc as plsc

_NC, _NS = 2, 16
_CW = 125
_P = 8


def _lin0_body(x_ref, w_ref, b_ref, o_ref):
    y = jnp.dot(x_ref[...], w_ref[...], preferred_element_type=jnp.float32)
    o_ref[...] = jnp.maximum(y + b_ref[...], 0.0)


def _lin0(x8, w8, b8):
    n8, d8 = x8.shape
    hp = w8.shape[1]
    return pl.pallas_call(
        _lin0_body,
        grid=(1,),
        in_specs=[
            pl.BlockSpec((n8, d8), lambda i: (0, 0)),
            pl.BlockSpec((d8, hp), lambda i: (0, 0)),
            pl.BlockSpec((1, hp), lambda i: (0, 0)),
        ],
        out_specs=pl.BlockSpec((n8, hp), lambda i: (0, 0)),
        out_shape=jax.ShapeDtypeStruct((n8, hp), jnp.float32),
    )(x8, w8, b8)


def _edge_body(xj_ref, ea_ref, g8_ref, r8_ref, s_ref, b8_ref, o_ref):
    xj = xj_ref[...]
    y = jnp.dot(xj, g8_ref[...], preferred_element_type=jnp.float32)
    er = jnp.dot(ea_ref[...], r8_ref[...],
                 preferred_element_type=jnp.float32)
    prod = y * er
    s = s_ref[...]
    hh = s.shape[0]
    parts = [
        jnp.dot(prod[:, k * hh:(k + 1) * hh], s,
                preferred_element_type=jnp.float32)
        for k in range(_P)
    ]
    msg = jnp.concatenate(parts, axis=1)
    o_ref[...] = msg + jnp.dot(xj, b8_ref[...],
                               preferred_element_type=jnp.float32)


def _edge_msg(xj8, ea8, g8, r8, smat, b8):
    e8, hp = xj8.shape
    w8 = g8.shape[1]
    hh = smat.shape[0]
    be8 = 1000
    return pl.pallas_call(
        _edge_body,
        grid=(e8 // be8,),
        in_specs=[
            pl.BlockSpec((be8, hp), lambda i: (i, 0)),
            pl.BlockSpec((be8, hp), lambda i: (i, 0)),
            pl.BlockSpec((hp, w8), lambda i: (0, 0)),
            pl.BlockSpec((hp, w8), lambda i: (0, 0)),
            pl.BlockSpec((hh, hp // _P), lambda i: (0, 0)),
            pl.BlockSpec((hp, hp), lambda i: (0, 0)),
        ],
        out_specs=pl.BlockSpec((be8, hp), lambda i: (i, 0)),
        out_shape=jax.ShapeDtypeStruct((e8, hp), jnp.float32),
    )(xj8, ea8, g8, r8, smat, b8)


def _node_body(agg2a_ref, agg2b_ref, h_ref, rwt_ref, cb_ref, wir_ref,
               wiz_ref, win_ref, whr_ref, whz_ref, whn_ref, bir_ref,
               biz_ref, bin_ref, bhr_ref, bhz_ref, bhn_ref, o_ref):
    h = h_ref[...]
    agg = ((agg2a_ref[0] + agg2a_ref[1])
           + (agg2b_ref[0] + agg2b_ref[1]))
    m = agg + jnp.dot(h, rwt_ref[...], preferred_element_type=jnp.float32)
    m = jnp.maximum(m + cb_ref[...], 0.0)

    def mm(a, w_ref):
        return jnp.dot(a, w_ref[...], preferred_element_type=jnp.float32)

    r = 1.0 / (1.0 + jnp.exp(-(mm(m, wir_ref) + bir_ref[...]
                               + mm(h, whr_ref) + bhr_ref[...])))
    z = 1.0 / (1.0 + jnp.exp(-(mm(m, wiz_ref) + biz_ref[...]
                               + mm(h, whz_ref) + bhz_ref[...])))
    nn = jnp.tanh(mm(m, win_ref) + bin_ref[...]
                  + r * (mm(h, whn_ref) + bhn_ref[...]))
    o_ref[...] = (1.0 - z) * nn + z * h


def _node_update(agg28a, agg28b, h8, rwt8, cb8, gate_ws, gate_bs):
    n8, hp = h8.shape
    wspec = pl.BlockSpec((hp, hp), lambda i: (0, 0))
    bspec = pl.BlockSpec((1, hp), lambda i: (0, 0))
    return pl.pallas_call(
        _node_body,
        grid=(1,),
        in_specs=[
            pl.BlockSpec((_NC, n8, hp), lambda i: (0, 0, 0)),
            pl.BlockSpec((_NC, n8, hp), lambda i: (0, 0, 0)),
            pl.BlockSpec((n8, hp), lambda i: (0, 0)),
            wspec, bspec,
            wspec, wspec, wspec, wspec, wspec, wspec,
            bspec, bspec, bspec, bspec, bspec, bspec,
        ],
        out_specs=pl.BlockSpec((n8, hp), lambda i: (0, 0)),
        out_shape=jax.ShapeDtypeStruct((n8, hp), jnp.float32),
    )(agg28a, agg28b, h8, rwt8, cb8, *gate_ws, *gate_bs)


def _sc_mesh():
    return plsc.VectorSubcoreMesh(core_axis_name="c", subcore_axis_name="s",
                                  num_cores=_NC, num_subcores=_NS)


def _sc_gather(table, src3):
    nw, ch, cw = src3.shape
    epw = ch * cw
    e = nw * epw
    h = table.shape[1]

    @functools.partial(
        pl.kernel,
        mesh=_sc_mesh(),
        out_type=jax.ShapeDtypeStruct((e, h), jnp.float32),
        compiler_params=pltpu.CompilerParams(use_tc_tiling_on_sc=False),
        scratch_types=[
            pltpu.VMEM((ch, cw), jnp.int32),
            pltpu.VMEM((epw, h), jnp.float32),
            pltpu.SemaphoreType.DMA,
        ],
    )
    def k(table_hbm, idx_hbm, out_hbm, idx_v, rows_v, sem):
        w = lax.axis_index("c") * _NS + lax.axis_index("s")
        pltpu.sync_copy(idx_hbm.at[w], idx_v)

        def fire(j, carry):
            pltpu.async_copy(table_hbm.at[idx_v.at[j]],
                             rows_v.at[pl.ds(j * cw, cw)], sem)
            return carry

        def drain(j, carry):
            pltpu.make_async_copy(table_hbm.at[idx_v.at[j]],
                                  rows_v.at[pl.ds(j * cw, cw)], sem).wait()
            return carry

        lax.fori_loop(0, ch, fire, 0)
        lax.fori_loop(0, ch, drain, 0)
        pltpu.sync_copy(rows_v, out_hbm.at[pl.ds(w * epw, epw)])

    return k(table, src3)


def _sc_scatter(msg, dst3, zeros, n):
    nw, ch, cw = dst3.shape
    epw = ch * cw
    h = msg.shape[1]
    stripe = n // _NS

    @functools.partial(
        pl.kernel,
        mesh=_sc_mesh(),
        out_type=jax.ShapeDtypeStruct((_NC, n, h), jnp.float32),
        compiler_params=pltpu.CompilerParams(use_tc_tiling_on_sc=False),
        scratch_types=[
            pltpu.VMEM((ch, cw), jnp.int32),
            pltpu.VMEM((epw, h), jnp.float32),
            pltpu.VMEM_SHARED((n, h), jnp.float32),
        ],
    )
    def k(msg_hbm, idx_hbm, z_hbm, out_hbm, idx_v, msg_v, agg_sh):
        cid = lax.axis_index("c")
        sid = lax.axis_index("s")
        w = cid * _NS + sid
        row0 = sid * stripe
        pltpu.sync_copy(z_hbm.at[pl.ds(row0, stripe)],
                        agg_sh.at[pl.ds(row0, stripe)])
        pltpu.sync_copy(idx_hbm.at[w], idx_v)
        pltpu.sync_copy(msg_hbm.at[pl.ds(w * epw, epw)], msg_v)
        plsc.subcore_barrier()

        def body(j, carry):
            pltpu.sync_copy(msg_v.at[pl.ds(j * cw, cw)],
                            agg_sh.at[idx_v.at[j]], add=True)
            return carry

        lax.fori_loop(0, ch, body, 0)
        plsc.subcore_barrier()
        pltpu.sync_copy(agg_sh.at[pl.ds(row0, stripe)],
                        out_hbm.at[cid, pl.ds(row0, stripe)])

    return k(msg, dst3, zeros)


def kernel(x, edge_index, edge_attr, lin0_W, lin0_b, nn_W, nn_b, root_W,
           conv_b, W_ih, W_hh, b_ih, b_hh):
    n, ind = x.shape
    e = edge_index.shape[1]
    h = lin0_W.shape[0]
    ed = edge_attr.shape[1]
    hp = _P * h
    nw = _NC * _NS
    eh = e // 2
    epw = eh // nw
    ch = epw // _CW
    eye8 = jnp.eye(_P, dtype=jnp.float32)

    src3 = edge_index[0].reshape(2, nw, ch, _CW)
    dst3 = edge_index[1].reshape(2, nw, ch, _CW)

    gcat = nn_W.reshape(h, h, ed).transpose(0, 2, 1).reshape(h, ed * h)
    rmat = jnp.repeat(jnp.eye(ed, dtype=jnp.float32), h, axis=1)
    smat = jnp.tile(jnp.eye(h, dtype=jnp.float32), (ed, 1))
    g8 = jnp.kron(eye8, gcat)
    r8 = jnp.kron(eye8, rmat)
    b8 = jnp.kron(eye8, nn_b.reshape(h, h))
    zeros = jnp.zeros((n, h), jnp.float32)
    ea8 = edge_attr.reshape(2, eh // _P, hp)

    l8 = jnp.kron(eye8, lin0_W.T)
    bl8 = jnp.tile(lin0_b, _P).reshape(1, hp)
    rwt8 = jnp.kron(eye8, root_W.T)
    cb8 = jnp.tile(conv_b, _P).reshape(1, hp)
    gate_ws = [jnp.kron(eye8, wmat[i * h:(i + 1) * h].T)
               for wmat in (W_ih, W_hh) for i in range(3)]
    gate_bs = [jnp.tile(bvec[i * h:(i + 1) * h], _P).reshape(1, hp)
               for bvec in (b_ih, b_hh) for i in range(3)]

    out8 = _lin0(x.reshape(n // _P, _P * ind), l8, bl8)
    h8 = out8
    for _ in range(3):
        table = out8.reshape(n, h)
        xj_a = _sc_gather(table, src3[0])
        xj_b = _sc_gather(table, src3[1])
        msg8a = _edge_msg(xj_a.reshape(eh // _P, hp), ea8[0], g8, r8, smat, b8)
        agg2a = _sc_scatter(msg8a.reshape(eh, h), dst3[0], zeros, n)
        msg8b = _edge_msg(xj_b.reshape(eh // _P, hp), ea8[1], g8, r8, smat, b8)
        agg2b = _sc_scatter(msg8b.reshape(eh, h), dst3[1], zeros, n)
        h8 = _node_update(agg2a.reshape(_NC, n // _P, hp),
                          agg2b.reshape(_NC, n // _P, hp), h8, rwt8, cb8,
                          gate_ws, gate_bs)
        out8 = h8
    return out8.reshape(n, h)


# speedup vs baseline: 1.1636x; 1.1636x over previous
import functools

import jax
import jax.numpy as jnp
from jax import lax
from jax.experimental import pallas as pl
from jax.experimental.pallas import tpu as pltpu
from jax.experimental.pallas import tpu_sc as plsc

_NC, _NS = 2, 16
_CW = 125
_P = 8


def _lin0_body(x_ref, w_ref, b_ref, o_ref):
    y = jnp.dot(x_ref[...], w_ref[...], preferred_element_type=jnp.float32)
    o_ref[...] = jnp.maximum(y + b_ref[...], 0.0)


def _lin0(x8, w8, b8):
    n8, d8 = x8.shape
    hp = w8.shape[1]
    return pl.pallas_call(
        _lin0_body,
        grid=(1,),
        in_specs=[
            pl.BlockSpec((n8, d8), lambda i: (0, 0)),
            pl.BlockSpec((d8, hp), lambda i: (0, 0)),
            pl.BlockSpec((1, hp), lambda i: (0, 0)),
        ],
        out_specs=pl.BlockSpec((n8, hp), lambda i: (0, 0)),
        out_shape=jax.ShapeDtypeStruct((n8, hp), jnp.float32),
    )(x8, w8, b8)


def _edge_body(xj_ref, ea_ref, g8_ref, r8_ref, s_ref, b8_ref, o_ref):
    xj = xj_ref[...]
    y = jnp.dot(xj, g8_ref[...], preferred_element_type=jnp.float32)
    er = jnp.dot(ea_ref[...], r8_ref[...],
                 preferred_element_type=jnp.float32)
    prod = y * er
    s = s_ref[...]
    hh = s.shape[0]
    parts = [
        jnp.dot(prod[:, k * hh:(k + 1) * hh], s,
                preferred_element_type=jnp.float32)
        for k in range(_P)
    ]
    msg = jnp.concatenate(parts, axis=1)
    o_ref[...] = msg + jnp.dot(xj, b8_ref[...],
                               preferred_element_type=jnp.float32)


def _edge_msg(xj8, ea8, g8, r8, smat, b8):
    e8, hp = xj8.shape
    w8 = g8.shape[1]
    hh = smat.shape[0]
    be8 = 1000
    return pl.pallas_call(
        _edge_body,
        grid=(e8 // be8,),
        in_specs=[
            pl.BlockSpec((be8, hp), lambda i: (i, 0)),
            pl.BlockSpec((be8, hp), lambda i: (i, 0)),
            pl.BlockSpec((hp, w8), lambda i: (0, 0)),
            pl.BlockSpec((hp, w8), lambda i: (0, 0)),
            pl.BlockSpec((hh, hp // _P), lambda i: (0, 0)),
            pl.BlockSpec((hp, hp), lambda i: (0, 0)),
        ],
        out_specs=pl.BlockSpec((be8, hp), lambda i: (i, 0)),
        out_shape=jax.ShapeDtypeStruct((e8, hp), jnp.float32),
    )(xj8, ea8, g8, r8, smat, b8)


def _node_body(agg2_ref, h_ref, rwt_ref, cb_ref, wir_ref,
               wiz_ref, win_ref, whr_ref, whz_ref, whn_ref, bir_ref,
               biz_ref, bin_ref, bhr_ref, bhz_ref, bhn_ref, o_ref):
    h = h_ref[...]
    agg = agg2_ref[0] + agg2_ref[1]
    m = agg + jnp.dot(h, rwt_ref[...], preferred_element_type=jnp.float32)
    m = jnp.maximum(m + cb_ref[...], 0.0)

    def mm(a, w_ref):
        return jnp.dot(a, w_ref[...], preferred_element_type=jnp.float32)

    r = 1.0 / (1.0 + jnp.exp(-(mm(m, wir_ref) + bir_ref[...]
                               + mm(h, whr_ref) + bhr_ref[...])))
    z = 1.0 / (1.0 + jnp.exp(-(mm(m, wiz_ref) + biz_ref[...]
                               + mm(h, whz_ref) + bhz_ref[...])))
    nn = jnp.tanh(mm(m, win_ref) + bin_ref[...]
                  + r * (mm(h, whn_ref) + bhn_ref[...]))
    o_ref[...] = (1.0 - z) * nn + z * h


def _node_update(agg28, h8, rwt8, cb8, gate_ws, gate_bs):
    n8, hp = h8.shape
    wspec = pl.BlockSpec((hp, hp), lambda i: (0, 0))
    bspec = pl.BlockSpec((1, hp), lambda i: (0, 0))
    return pl.pallas_call(
        _node_body,
        grid=(1,),
        in_specs=[
            pl.BlockSpec((_NC, n8, hp), lambda i: (0, 0, 0)),
            pl.BlockSpec((n8, hp), lambda i: (0, 0)),
            wspec, bspec,
            wspec, wspec, wspec, wspec, wspec, wspec,
            bspec, bspec, bspec, bspec, bspec, bspec,
        ],
        out_specs=pl.BlockSpec((n8, hp), lambda i: (0, 0)),
        out_shape=jax.ShapeDtypeStruct((n8, hp), jnp.float32),
    )(agg28, h8, rwt8, cb8, *gate_ws, *gate_bs)


def _sc_mesh():
    return plsc.VectorSubcoreMesh(core_axis_name="c", subcore_axis_name="s",
                                  num_cores=_NC, num_subcores=_NS)


def _sc_gather(table, src3):
    nw, ch, cw = src3.shape
    epw = ch * cw
    e = nw * epw
    h = table.shape[1]

    @functools.partial(
        pl.kernel,
        mesh=_sc_mesh(),
        out_type=jax.ShapeDtypeStruct((e, h), jnp.float32),
        compiler_params=pltpu.CompilerParams(use_tc_tiling_on_sc=False),
        scratch_types=[
            pltpu.VMEM((ch, cw), jnp.int32),
            pltpu.VMEM((epw, h), jnp.float32),
            pltpu.SemaphoreType.DMA,
        ],
    )
    def k(table_hbm, idx_hbm, out_hbm, idx_v, rows_v, sem):
        w = lax.axis_index("c") * _NS + lax.axis_index("s")
        pltpu.sync_copy(idx_hbm.at[w], idx_v)

        def fire(j, carry):
            pltpu.async_copy(table_hbm.at[idx_v.at[j]],
                             rows_v.at[pl.ds(j * cw, cw)], sem)
            return carry

        def drain(j, carry):
            pltpu.make_async_copy(table_hbm.at[idx_v.at[j]],
                                  rows_v.at[pl.ds(j * cw, cw)], sem).wait()
            return carry

        lax.fori_loop(0, ch, fire, 0)
        lax.fori_loop(0, ch, drain, 0)
        pltpu.sync_copy(rows_v, out_hbm.at[pl.ds(w * epw, epw)])

    return k(table, src3)


def _sc_scatter(msg, dst3, zeros, n):
    nw, ch, cw = dst3.shape
    epw = ch * cw
    h = msg.shape[1]
    stripe = n // _NS

    @functools.partial(
        pl.kernel,
        mesh=_sc_mesh(),
        out_type=jax.ShapeDtypeStruct((_NC, n, h), jnp.float32),
        compiler_params=pltpu.CompilerParams(use_tc_tiling_on_sc=False),
        scratch_types=[
            pltpu.VMEM((ch, cw), jnp.int32),
            pltpu.VMEM((epw, h), jnp.float32),
            pltpu.VMEM_SHARED((n, h), jnp.float32),
            pltpu.SemaphoreType.DMA,
        ],
    )
    def k(msg_hbm, idx_hbm, z_hbm, out_hbm, idx_v, msg_v, agg_sh, sem):
        cid = lax.axis_index("c")
        sid = lax.axis_index("s")
        w = cid * _NS + sid
        row0 = sid * stripe
        pltpu.sync_copy(z_hbm.at[pl.ds(row0, stripe)],
                        agg_sh.at[pl.ds(row0, stripe)])
        pltpu.sync_copy(idx_hbm.at[w], idx_v)
        pltpu.sync_copy(msg_hbm.at[pl.ds(w * epw, epw)], msg_v)
        plsc.subcore_barrier()

        def fire(j, carry):
            pltpu.async_copy(msg_v.at[pl.ds(j * cw, cw)],
                             agg_sh.at[idx_v.at[j]], sem, add=True)
            return carry

        def drain(j, carry):
            pltpu.make_async_copy(msg_v.at[pl.ds(j * cw, cw)],
                                  agg_sh.at[idx_v.at[j]], sem).wait()
            return carry

        lax.fori_loop(0, ch, fire, 0)
        lax.fori_loop(0, ch, drain, 0)
        plsc.subcore_barrier()
        pltpu.sync_copy(agg_sh.at[pl.ds(row0, stripe)],
                        out_hbm.at[cid, pl.ds(row0, stripe)])

    return k(msg, dst3, zeros)


def kernel(x, edge_index, edge_attr, lin0_W, lin0_b, nn_W, nn_b, root_W,
           conv_b, W_ih, W_hh, b_ih, b_hh):
    n, ind = x.shape
    e = edge_index.shape[1]
    h = lin0_W.shape[0]
    ed = edge_attr.shape[1]
    hp = _P * h
    nw = _NC * _NS
    epw = e // nw
    ch = epw // _CW
    eye8 = jnp.eye(_P, dtype=jnp.float32)

    src3 = edge_index[0].reshape(nw, ch, _CW)
    dst3 = edge_index[1].reshape(nw, ch, _CW)

    gcat = nn_W.reshape(h, h, ed).transpose(0, 2, 1).reshape(h, ed * h)
    rmat = jnp.repeat(jnp.eye(ed, dtype=jnp.float32), h, axis=1)
    smat = jnp.tile(jnp.eye(h, dtype=jnp.float32), (ed, 1))
    g8 = jnp.kron(eye8, gcat)
    r8 = jnp.kron(eye8, rmat)
    b8 = jnp.kron(eye8, nn_b.reshape(h, h))
    zeros = jnp.zeros((n, h), jnp.float32)
    ea8 = edge_attr.reshape(e // _P, hp)

    l8 = jnp.kron(eye8, lin0_W.T)
    bl8 = jnp.tile(lin0_b, _P).reshape(1, hp)
    rwt8 = jnp.kron(eye8, root_W.T)
    cb8 = jnp.tile(conv_b, _P).reshape(1, hp)
    gate_ws = [jnp.kron(eye8, wmat[i * h:(i + 1) * h].T)
               for wmat in (W_ih, W_hh) for i in range(3)]
    gate_bs = [jnp.tile(bvec[i * h:(i + 1) * h], _P).reshape(1, hp)
               for bvec in (b_ih, b_hh) for i in range(3)]

    out8 = _lin0(x.reshape(n // _P, _P * ind), l8, bl8)
    h8 = out8
    for _ in range(3):
        xj = _sc_gather(out8.reshape(n, h), src3)
        msg8 = _edge_msg(xj.reshape(e // _P, hp), ea8, g8, r8, smat, b8)
        agg2 = _sc_scatter(msg8.reshape(e, h), dst3, zeros, n)
        h8 = _node_update(agg2.reshape(_NC, n // _P, hp), h8, rwt8, cb8,
                          gate_ws, gate_bs)
        out8 = h8
    return out8.reshape(n, h)


# speedup vs baseline: 1.1816x; 1.0155x over previous
import functools

import jax
import jax.numpy as jnp
from jax import lax
from jax.experimental import pallas as pl
from jax.experimental.pallas import tpu as pltpu
from jax.experimental.pallas import tpu_sc as plsc

_NC, _NS = 2, 16
_CW = 125
_P = 8


def _lin0_body(x_ref, w_ref, b_ref, o_ref):
    y = jnp.dot(x_ref[...], w_ref[...], preferred_element_type=jnp.float32)
    o_ref[...] = jnp.maximum(y + b_ref[...], 0.0)


def _lin0(x8, w8, b8):
    n8, d8 = x8.shape
    hp = w8.shape[1]
    return pl.pallas_call(
        _lin0_body,
        grid=(1,),
        in_specs=[
            pl.BlockSpec((n8, d8), lambda i: (0, 0)),
            pl.BlockSpec((d8, hp), lambda i: (0, 0)),
            pl.BlockSpec((1, hp), lambda i: (0, 0)),
        ],
        out_specs=pl.BlockSpec((n8, hp), lambda i: (0, 0)),
        out_shape=jax.ShapeDtypeStruct((n8, hp), jnp.float32),
    )(x8, w8, b8)


def _edge_body(xj_ref, ea_ref, g8_ref, r8_ref, s_ref, b8_ref, o_ref):
    xj = xj_ref[...]
    y = jnp.dot(xj, g8_ref[...], preferred_element_type=jnp.float32)
    er = jnp.dot(ea_ref[...], r8_ref[...],
                 preferred_element_type=jnp.float32)
    prod = y * er
    s = s_ref[...]
    hh = s.shape[0]
    parts = [
        jnp.dot(prod[:, k * hh:(k + 1) * hh], s,
                preferred_element_type=jnp.float32)
        for k in range(_P)
    ]
    msg = jnp.concatenate(parts, axis=1)
    o_ref[...] = msg + jnp.dot(xj, b8_ref[...],
                               preferred_element_type=jnp.float32)


def _edge_msg(xj8, ea8, g8, r8, smat, b8):
    e8, hp = xj8.shape
    w8 = g8.shape[1]
    hh = smat.shape[0]
    be8 = 2000
    return pl.pallas_call(
        _edge_body,
        grid=(e8 // be8,),
        in_specs=[
            pl.BlockSpec((be8, hp), lambda i: (i, 0)),
            pl.BlockSpec((be8, hp), lambda i: (i, 0)),
            pl.BlockSpec((hp, w8), lambda i: (0, 0)),
            pl.BlockSpec((hp, w8), lambda i: (0, 0)),
            pl.BlockSpec((hh, hp // _P), lambda i: (0, 0)),
            pl.BlockSpec((hp, hp), lambda i: (0, 0)),
        ],
        out_specs=pl.BlockSpec((be8, hp), lambda i: (i, 0)),
        out_shape=jax.ShapeDtypeStruct((e8, hp), jnp.float32),
    )(xj8, ea8, g8, r8, smat, b8)


def _node_body(agg2_ref, h_ref, rwt_ref, cb_ref, wir_ref,
               wiz_ref, win_ref, whr_ref, whz_ref, whn_ref, bir_ref,
               biz_ref, bin_ref, bhr_ref, bhz_ref, bhn_ref, o_ref):
    h = h_ref[...]
    agg = agg2_ref[0] + agg2_ref[1]
    m = agg + jnp.dot(h, rwt_ref[...], preferred_element_type=jnp.float32)
    m = jnp.maximum(m + cb_ref[...], 0.0)

    def mm(a, w_ref):
        return jnp.dot(a, w_ref[...], preferred_element_type=jnp.float32)

    r = 1.0 / (1.0 + jnp.exp(-(mm(m, wir_ref) + bir_ref[...]
                               + mm(h, whr_ref) + bhr_ref[...])))
    z = 1.0 / (1.0 + jnp.exp(-(mm(m, wiz_ref) + biz_ref[...]
                               + mm(h, whz_ref) + bhz_ref[...])))
    nn = jnp.tanh(mm(m, win_ref) + bin_ref[...]
                  + r * (mm(h, whn_ref) + bhn_ref[...]))
    o_ref[...] = (1.0 - z) * nn + z * h


def _node_update(agg28, h8, rwt8, cb8, gate_ws, gate_bs):
    n8, hp = h8.shape
    wspec = pl.BlockSpec((hp, hp), lambda i: (0, 0))
    bspec = pl.BlockSpec((1, hp), lambda i: (0, 0))
    return pl.pallas_call(
        _node_body,
        grid=(1,),
        in_specs=[
            pl.BlockSpec((_NC, n8, hp), lambda i: (0, 0, 0)),
            pl.BlockSpec((n8, hp), lambda i: (0, 0)),
            wspec, bspec,
            wspec, wspec, wspec, wspec, wspec, wspec,
            bspec, bspec, bspec, bspec, bspec, bspec,
        ],
        out_specs=pl.BlockSpec((n8, hp), lambda i: (0, 0)),
        out_shape=jax.ShapeDtypeStruct((n8, hp), jnp.float32),
    )(agg28, h8, rwt8, cb8, *gate_ws, *gate_bs)


def _sc_mesh():
    return plsc.VectorSubcoreMesh(core_axis_name="c", subcore_axis_name="s",
                                  num_cores=_NC, num_subcores=_NS)


def _sc_gather(table, src3):
    nw, ch, cw = src3.shape
    epw = ch * cw
    e = nw * epw
    h = table.shape[1]

    @functools.partial(
        pl.kernel,
        mesh=_sc_mesh(),
        out_type=jax.ShapeDtypeStruct((e, h), jnp.float32),
        compiler_params=pltpu.CompilerParams(use_tc_tiling_on_sc=False),
        scratch_types=[
            pltpu.VMEM((ch, cw), jnp.int32),
            pltpu.VMEM((epw, h), jnp.float32),
            pltpu.SemaphoreType.DMA,
        ],
    )
    def k(table_hbm, idx_hbm, out_hbm, idx_v, rows_v, sem):
        w = lax.axis_index("c") * _NS + lax.axis_index("s")
        pltpu.sync_copy(idx_hbm.at[w], idx_v)

        def fire(j, carry):
            pltpu.async_copy(table_hbm.at[idx_v.at[j]],
                             rows_v.at[pl.ds(j * cw, cw)], sem)
            return carry

        def drain(j, carry):
            pltpu.make_async_copy(table_hbm.at[idx_v.at[j]],
                                  rows_v.at[pl.ds(j * cw, cw)], sem).wait()
            return carry

        lax.fori_loop(0, ch, fire, 0)
        lax.fori_loop(0, ch, drain, 0)
        pltpu.sync_copy(rows_v, out_hbm.at[pl.ds(w * epw, epw)])

    return k(table, src3)


def _sc_scatter(msg, dst3, zeros, n):
    nw, ch, cw = dst3.shape
    epw = ch * cw
    h = msg.shape[1]
    stripe = n // _NS

    @functools.partial(
        pl.kernel,
        mesh=_sc_mesh(),
        out_type=jax.ShapeDtypeStruct((_NC, n, h), jnp.float32),
        compiler_params=pltpu.CompilerParams(use_tc_tiling_on_sc=False),
        scratch_types=[
            pltpu.VMEM((ch, cw), jnp.int32),
            pltpu.VMEM((epw, h), jnp.float32),
            pltpu.VMEM_SHARED((n, h), jnp.float32),
            pltpu.SemaphoreType.DMA,
        ],
    )
    def k(msg_hbm, idx_hbm, z_hbm, out_hbm, idx_v, msg_v, agg_sh, sem):
        cid = lax.axis_index("c")
        sid = lax.axis_index("s")
        w = cid * _NS + sid
        row0 = sid * stripe
        pltpu.sync_copy(z_hbm.at[pl.ds(row0, stripe)],
                        agg_sh.at[pl.ds(row0, stripe)])
        pltpu.sync_copy(idx_hbm.at[w], idx_v)
        pltpu.sync_copy(msg_hbm.at[pl.ds(w * epw, epw)], msg_v)
        plsc.subcore_barrier()

        def fire(j, carry):
            pltpu.async_copy(msg_v.at[pl.ds(j * cw, cw)],
                             agg_sh.at[idx_v.at[j]], sem, add=True)
            return carry

        def drain(j, carry):
            pltpu.make_async_copy(msg_v.at[pl.ds(j * cw, cw)],
                                  agg_sh.at[idx_v.at[j]], sem).wait()
            return carry

        lax.fori_loop(0, ch, fire, 0)
        lax.fori_loop(0, ch, drain, 0)
        plsc.subcore_barrier()
        pltpu.sync_copy(agg_sh.at[pl.ds(row0, stripe)],
                        out_hbm.at[cid, pl.ds(row0, stripe)])

    return k(msg, dst3, zeros)


def kernel(x, edge_index, edge_attr, lin0_W, lin0_b, nn_W, nn_b, root_W,
           conv_b, W_ih, W_hh, b_ih, b_hh):
    n, ind = x.shape
    e = edge_index.shape[1]
    h = lin0_W.shape[0]
    ed = edge_attr.shape[1]
    hp = _P * h
    nw = _NC * _NS
    epw = e // nw
    ch = epw // _CW
    eye8 = jnp.eye(_P, dtype=jnp.float32)

    src3 = edge_index[0].reshape(nw, ch, _CW)
    dst3 = edge_index[1].reshape(nw, ch, _CW)

    gcat = nn_W.reshape(h, h, ed).transpose(0, 2, 1).reshape(h, ed * h)
    rmat = jnp.repeat(jnp.eye(ed, dtype=jnp.float32), h, axis=1)
    smat = jnp.tile(jnp.eye(h, dtype=jnp.float32), (ed, 1))
    g8 = jnp.kron(eye8, gcat)
    r8 = jnp.kron(eye8, rmat)
    b8 = jnp.kron(eye8, nn_b.reshape(h, h))
    zeros = jnp.zeros((n, h), jnp.float32)
    ea8 = edge_attr.reshape(e // _P, hp)

    l8 = jnp.kron(eye8, lin0_W.T)
    bl8 = jnp.tile(lin0_b, _P).reshape(1, hp)
    rwt8 = jnp.kron(eye8, root_W.T)
    cb8 = jnp.tile(conv_b, _P).reshape(1, hp)
    gate_ws = [jnp.kron(eye8, wmat[i * h:(i + 1) * h].T)
               for wmat in (W_ih, W_hh) for i in range(3)]
    gate_bs = [jnp.tile(bvec[i * h:(i + 1) * h], _P).reshape(1, hp)
               for bvec in (b_ih, b_hh) for i in range(3)]

    out8 = _lin0(x.reshape(n // _P, _P * ind), l8, bl8)
    h8 = out8
    for _ in range(3):
        xj = _sc_gather(out8.reshape(n, h), src3)
        msg8 = _edge_msg(xj.reshape(e // _P, hp), ea8, g8, r8, smat, b8)
        agg2 = _sc_scatter(msg8.reshape(e, h), dst3, zeros, n)
        h8 = _node_update(agg2.reshape(_NC, n // _P, hp), h8, rwt8, cb8,
                          gate_ws, gate_bs)
        out8 = h8
    return out8.reshape(n, h)
